# baseline (speedup 1.0000x reference)
import functools

import jax
import jax.numpy as jnp
from jax import lax
from jax.experimental import pallas as pl
from jax.experimental.pallas import tpu as pltpu
from jax.experimental.pallas import tpu_sc as plsc

D_MODEL = 1024
STARTS = (0, 100000, 110000, 111000, 116000)
SIZES = (100000, 10000, 1000, 5000, 12000)
NUM_TABLES = 5

NC = 2
NS = 16
NW = NC * NS
L = 16
G = 16
NBUF = 4


@functools.partial(jax.jit, static_argnums=())
def _embed(ids_flat, text_w, audio_w, special_w, phoneme_w, reserved_w):
    B = ids_flat.shape[0]
    C = B // NW
    NV = C // L

    mesh = plsc.VectorSubcoreMesh(
        core_axis_name="c", subcore_axis_name="s", num_cores=NC,
        num_subcores=NS)

    scratch = (
        [pltpu.VMEM((C,), jnp.int32)]
        + [pltpu.VMEM((C + G,), jnp.int32) for _ in range(NUM_TABLES)]
        + [pltpu.VMEM((C + G,), jnp.int32) for _ in range(NUM_TABLES)]
        + [pltpu.VMEM((G,), jnp.int32) for _ in range(NBUF)]
        + [pltpu.VMEM((G, D_MODEL), jnp.float32) for _ in range(NBUF)]
        + [pltpu.SemaphoreType.DMA for _ in range(2 * NBUF)]
    )

    @functools.partial(
        pl.kernel,
        out_type=jax.ShapeDtypeStruct((B, D_MODEL), jnp.float32),
        mesh=mesh,
        scratch_types=scratch,
        compiler_params=pltpu.CompilerParams(
            needs_layout_passes=False, use_tc_tiling_on_sc=False),
    )
    def body(ids_hbm, t0, t1, t2, t3, t4, out_hbm, ids_v, *refs):
        tables = (t0, t1, t2, t3, t4)
        refs = list(refs)
        idx_bufs = tuple(refs[:NUM_TABLES]); del refs[:NUM_TABLES]
        pos_bufs = tuple(refs[:NUM_TABLES]); del refs[:NUM_TABLES]
        pos_gs = tuple(refs[:NBUF]); del refs[:NBUF]
        rows = tuple(refs[:NBUF]); del refs[:NBUF]
        gsems = tuple(refs[:NBUF]); del refs[:NBUF]
        ssems = tuple(refs[:NBUF]); del refs[:NBUF]

        wid = lax.axis_index("s") * NC + lax.axis_index("c")
        base = wid * C
        pltpu.sync_copy(ids_hbm.at[pl.ds(base, C)], ids_v)
        lane = lax.iota(jnp.int32, L)

        def cbody(i, counts_v):
            v = ids_v[pl.ds(i * L, L)]
            pos = base + i * L + lane
            new = []
            for t in range(NUM_TABLES):
                m = (v >= STARTS[t]) & (v < STARTS[t] + SIZES[t])
                mc = plsc.cumsum(m.astype(jnp.int32))
                offs = jnp.maximum(counts_v[t] + mc - 1, 0)
                plsc.store_scatter(
                    idx_bufs[t], [offs], v - STARTS[t], mask=m)
                plsc.store_scatter(pos_bufs[t], [offs], pos, mask=m)
                new.append(counts_v[t] + jnp.full((L,), mc[L - 1], jnp.int32))
            return tuple(new)

        counts_v = lax.fori_loop(
            0, NV, cbody,
            tuple(jnp.zeros((L,), jnp.int32) for _ in range(NUM_TABLES)))
        counts = tuple(counts_v[t][0] for t in range(NUM_TABLES))

        for t in range(NUM_TABLES):
            cnt = counts[t]
            last = jnp.maximum(cnt - 1, 0)
            lastv = idx_bufs[t][pl.ds(last, L)][0]
            lastp = pos_bufs[t][pl.ds(last, L)][0]
            for j in range(G // L):
                idx_bufs[t][pl.ds(cnt + j * L, L)] = jnp.full(
                    (L,), lastv, jnp.int32)
                pos_bufs[t][pl.ds(cnt + j * L, L)] = jnp.full(
                    (L,), lastp, jnp.int32)

        ngs = [(counts[t] + (G - 1)) // G for t in range(NUM_TABLES)]
        cums = []
        acc = jnp.int32(0)
        for t in range(NUM_TABLES):
            acc = acc + ngs[t]
            cums.append(acc)
        ngt = cums[-1]

        def start_group(b, g):
            t_sel = jnp.int32(0)
            for t in range(NUM_TABLES - 1):
                t_sel = t_sel + (g >= cums[t]).astype(jnp.int32)

            def mk_branch(t):
                def br(b=b, t=t):
                    gl = g - (cums[t - 1] if t > 0 else 0)
                    for j in range(G // L):
                        pos_gs[b][pl.ds(j * L, L)] = (
                            pos_bufs[t][pl.ds(gl * G + j * L, L)])
                    pltpu.make_async_copy(
                        tables[t].at[idx_bufs[t].at[pl.ds(gl * G, G)]],
                        rows[b], gsems[b]).start()
                return br

            lax.switch(t_sel, [mk_branch(t) for t in range(NUM_TABLES)])

        def gwait(b):
            pltpu.make_async_copy(
                tables[0].at[idx_bufs[0].at[pl.ds(0, G)]],
                rows[b], gsems[b]).wait()

        def scopy(b):
            return pltpu.make_async_copy(
                rows[b], out_hbm.at[pos_gs[b]], ssems[b])

        ns = (ngt + (NBUF - 1)) // NBUF

        def sbody(s, carry):
            for b in range(NBUF):
                g = s * NBUF + b

                @pl.when(s > 0)
                def _(b=b):
                    scopy(b).wait()

                @pl.when(g < ngt)
                def _(b=b, g=g):
                    start_group(b, g)

            for b in range(NBUF):
                g = s * NBUF + b

                @pl.when(g < ngt)
                def _(b=b):
                    gwait(b)
                    scopy(b).start()
            return carry

        lax.fori_loop(0, ns, sbody, jnp.int32(0))
        for b in range(NBUF):
            @pl.when((ns - 1) * NBUF + b < ngt)
            def _(b=b):
                scopy(b).wait()

    return body(ids_flat, text_w, audio_w, special_w, phoneme_w, reserved_w)


def kernel(input_ids, text_w, audio_w, special_w, phoneme_w, reserved_w):
    shape = input_ids.shape
    ids_flat = input_ids.reshape(-1).astype(jnp.int32)
    out = _embed(ids_flat, text_w, audio_w, special_w, phoneme_w, reserved_w)
    return out.reshape(*shape, D_MODEL)


# speedup vs baseline: 5.0068x; 5.0068x over previous
import functools

import jax
import jax.numpy as jnp
from jax import lax
from jax.experimental import pallas as pl
from jax.experimental.pallas import tpu as pltpu
from jax.experimental.pallas import tpu_sc as plsc

D_MODEL = 1024
STARTS = (0, 100000, 110000, 111000, 116000)
SIZES = (100000, 10000, 1000, 5000, 12000)
NUM_TABLES = 5

NC = 2
NS = 16
NW = NC * NS
L = 16
G = 16
NBUF = 4


@functools.partial(jax.jit, static_argnums=())
def _embed(ids_flat, text_w, audio_w, special_w, phoneme_w, reserved_w):
    B = ids_flat.shape[0]
    C = B // NW
    NV = C // L

    mesh = plsc.VectorSubcoreMesh(
        core_axis_name="c", subcore_axis_name="s", num_cores=NC,
        num_subcores=NS)

    scratch = (
        [pltpu.VMEM((C,), jnp.int32)]
        + [pltpu.VMEM((C + G,), jnp.int32) for _ in range(NUM_TABLES)]
        + [pltpu.VMEM((C + G,), jnp.int32) for _ in range(NUM_TABLES)]
        + [pltpu.VMEM((G,), jnp.int32) for _ in range(NBUF)]
        + [pltpu.VMEM((G, D_MODEL), jnp.float32) for _ in range(NBUF)]
        + [pltpu.SemaphoreType.DMA for _ in range(2 * NBUF)]
    )

    @functools.partial(
        pl.kernel,
        out_type=jax.ShapeDtypeStruct((B, D_MODEL), jnp.float32),
        mesh=mesh,
        scratch_types=scratch,
        compiler_params=pltpu.CompilerParams(needs_layout_passes=False),
    )
    def body(ids_hbm, t0, t1, t2, t3, t4, out_hbm, ids_v, *refs):
        tables = (t0, t1, t2, t3, t4)
        refs = list(refs)
        idx_bufs = tuple(refs[:NUM_TABLES]); del refs[:NUM_TABLES]
        pos_bufs = tuple(refs[:NUM_TABLES]); del refs[:NUM_TABLES]
        pos_gs = tuple(refs[:NBUF]); del refs[:NBUF]
        rows = tuple(refs[:NBUF]); del refs[:NBUF]
        gsems = tuple(refs[:NBUF]); del refs[:NBUF]
        ssems = tuple(refs[:NBUF]); del refs[:NBUF]

        wid = lax.axis_index("s") * NC + lax.axis_index("c")
        base = wid * C
        pltpu.sync_copy(ids_hbm.at[pl.ds(base, C)], ids_v)
        lane = lax.iota(jnp.int32, L)

        def cbody(i, counts_v):
            v = ids_v[pl.ds(i * L, L)]
            pos = base + i * L + lane
            new = []
            for t in range(NUM_TABLES):
                m = (v >= STARTS[t]) & (v < STARTS[t] + SIZES[t])
                mc = plsc.cumsum(m.astype(jnp.int32))
                offs = jnp.maximum(counts_v[t] + mc - 1, 0)
                plsc.store_scatter(
                    idx_bufs[t], [offs], v - STARTS[t], mask=m)
                plsc.store_scatter(pos_bufs[t], [offs], pos, mask=m)
                new.append(counts_v[t] + jnp.full((L,), mc[L - 1], jnp.int32))
            return tuple(new)

        counts_v = lax.fori_loop(
            0, NV, cbody,
            tuple(jnp.zeros((L,), jnp.int32) for _ in range(NUM_TABLES)))
        counts = tuple(counts_v[t][0] for t in range(NUM_TABLES))

        for t in range(NUM_TABLES):
            cnt = counts[t]
            last = jnp.maximum(cnt - 1, 0)
            lastv = idx_bufs[t][pl.ds(last, L)][0]
            lastp = pos_bufs[t][pl.ds(last, L)][0]
            for j in range(G // L):
                idx_bufs[t][pl.ds(cnt + j * L, L)] = jnp.full(
                    (L,), lastv, jnp.int32)
                pos_bufs[t][pl.ds(cnt + j * L, L)] = jnp.full(
                    (L,), lastp, jnp.int32)

        ngs = [(counts[t] + (G - 1)) // G for t in range(NUM_TABLES)]
        cums = []
        acc = jnp.int32(0)
        for t in range(NUM_TABLES):
            acc = acc + ngs[t]
            cums.append(acc)
        ngt = cums[-1]

        def start_group(b, g):
            t_sel = jnp.int32(0)
            for t in range(NUM_TABLES - 1):
                t_sel = t_sel + (g >= cums[t]).astype(jnp.int32)

            def mk_branch(t):
                def br(b=b, t=t):
                    gl = g - (cums[t - 1] if t > 0 else 0)
                    for j in range(G // L):
                        pos_gs[b][pl.ds(j * L, L)] = (
                            pos_bufs[t][pl.ds(gl * G + j * L, L)])
                    pltpu.make_async_copy(
                        tables[t].at[idx_bufs[t].at[pl.ds(gl * G, G)]],
                        rows[b], gsems[b]).start()
                return br

            lax.switch(t_sel, [mk_branch(t) for t in range(NUM_TABLES)])

        def gwait(b):
            pltpu.make_async_copy(
                tables[0].at[idx_bufs[0].at[pl.ds(0, G)]],
                rows[b], gsems[b]).wait()

        def scopy(b):
            return pltpu.make_async_copy(
                rows[b], out_hbm.at[pos_gs[b]], ssems[b])

        ns = (ngt + (NBUF - 1)) // NBUF

        def sbody(s, carry):
            for b in range(NBUF):
                g = s * NBUF + b

                @pl.when(s > 0)
                def _(b=b):
                    scopy(b).wait()

                @pl.when(g < ngt)
                def _(b=b, g=g):
                    start_group(b, g)

            for b in range(NBUF):
                g = s * NBUF + b

                @pl.when(g < ngt)
                def _(b=b):
                    gwait(b)
                    scopy(b).start()
            return carry

        lax.fori_loop(0, ns, sbody, jnp.int32(0))
        for b in range(NBUF):
            @pl.when((ns - 1) * NBUF + b < ngt)
            def _(b=b):
                scopy(b).wait()

    return body(ids_flat, text_w, audio_w, special_w, phoneme_w, reserved_w)


def kernel(input_ids, text_w, audio_w, special_w, phoneme_w, reserved_w):
    shape = input_ids.shape
    ids_flat = input_ids.reshape(-1).astype(jnp.int32)
    out = _embed(ids_flat, text_w, audio_w, special_w, phoneme_w, reserved_w)
    return out.reshape(*shape, D_MODEL)


# speedup vs baseline: 5.0557x; 1.0098x over previous
import functools

import jax
import jax.numpy as jnp
from jax import lax
from jax.experimental import pallas as pl
from jax.experimental.pallas import tpu as pltpu
from jax.experimental.pallas import tpu_sc as plsc

D_MODEL = 1024
STARTS = (0, 100000, 110000, 111000, 116000)
SIZES = (100000, 10000, 1000, 5000, 12000)
NUM_TABLES = 5

NC = 2
NS = 16
NW = NC * NS
L = 16
G = 16
NBUF = 4


@functools.partial(jax.jit, static_argnums=())
def _embed(ids_flat, text_w, audio_w, special_w, phoneme_w, reserved_w):
    B = ids_flat.shape[0]
    C = B // NW
    NV = C // L

    mesh = plsc.VectorSubcoreMesh(
        core_axis_name="c", subcore_axis_name="s", num_cores=NC,
        num_subcores=NS)

    scratch = (
        [pltpu.VMEM((C,), jnp.int32)]
        + [pltpu.VMEM((C + G,), jnp.int32) for _ in range(NUM_TABLES)]
        + [pltpu.VMEM((C + G,), jnp.int32) for _ in range(NUM_TABLES)]
        + [pltpu.VMEM((G,), jnp.int32) for _ in range(NBUF)]
        + [pltpu.VMEM((G,), jnp.int32) for _ in range(NBUF)]
        + [pltpu.VMEM((G, D_MODEL), jnp.float32) for _ in range(NBUF)]
        + [pltpu.SemaphoreType.DMA for _ in range(2 * NBUF)]
    )

    @functools.partial(
        pl.kernel,
        out_type=jax.ShapeDtypeStruct((B, D_MODEL), jnp.float32),
        mesh=mesh,
        scratch_types=scratch,
        compiler_params=pltpu.CompilerParams(needs_layout_passes=False),
    )
    def body(ids_hbm, t0, t1, t2, t3, t4, out_hbm, ids_v, *refs):
        tables = (t0, t1, t2, t3, t4)
        refs = list(refs)
        idx_bufs = tuple(refs[:NUM_TABLES]); del refs[:NUM_TABLES]
        pos_bufs = tuple(refs[:NUM_TABLES]); del refs[:NUM_TABLES]
        idx_gs = tuple(refs[:NBUF]); del refs[:NBUF]
        pos_gs = tuple(refs[:NBUF]); del refs[:NBUF]
        rows = tuple(refs[:NBUF]); del refs[:NBUF]
        gsems = tuple(refs[:NBUF]); del refs[:NBUF]
        ssems = tuple(refs[:NBUF]); del refs[:NBUF]

        wid = lax.axis_index("s") * NC + lax.axis_index("c")
        base = wid * C
        pltpu.sync_copy(ids_hbm.at[pl.ds(base, C)], ids_v)
        lane = lax.iota(jnp.int32, L)

        def cbody(i, counts_v):
            v = ids_v[pl.ds(i * L, L)]
            pos = base + i * L + lane
            new = []
            for t in range(NUM_TABLES):
                m = (v >= STARTS[t]) & (v < STARTS[t] + SIZES[t])
                mc = plsc.cumsum(m.astype(jnp.int32))
                offs = jnp.maximum(counts_v[t] + mc - 1, 0)
                plsc.store_scatter(
                    idx_bufs[t], [offs], v - STARTS[t], mask=m)
                plsc.store_scatter(pos_bufs[t], [offs], pos, mask=m)
                new.append(counts_v[t] + jnp.full((L,), mc[L - 1], jnp.int32))
            return tuple(new)

        counts_v = lax.fori_loop(
            0, NV, cbody,
            tuple(jnp.zeros((L,), jnp.int32) for _ in range(NUM_TABLES)))
        counts = tuple(counts_v[t][0] for t in range(NUM_TABLES))

        for t in range(NUM_TABLES):
            cnt = counts[t]
            last = jnp.maximum(cnt - 1, 0)
            lastv = idx_bufs[t][pl.ds(last, L)][0]
            lastp = pos_bufs[t][pl.ds(last, L)][0]
            for j in range(G // L):
                idx_bufs[t][pl.ds(cnt + j * L, L)] = jnp.full(
                    (L,), lastv, jnp.int32)
                pos_bufs[t][pl.ds(cnt + j * L, L)] = jnp.full(
                    (L,), lastp, jnp.int32)

        ngs = [(counts[t] + (G - 1)) // G for t in range(NUM_TABLES)]
        cums = []
        acc = jnp.int32(0)
        for t in range(NUM_TABLES):
            acc = acc + ngs[t]
            cums.append(acc)
        ngt = cums[-1]

        def start_group(b, g):
            t_sel = jnp.int32(0)
            for t in range(NUM_TABLES - 1):
                t_sel = t_sel + (g >= cums[t]).astype(jnp.int32)

            def mk_branch(t):
                def br(b=b, t=t):
                    gl = g - (cums[t - 1] if t > 0 else 0)
                    kv = plsc.sort_key_val(
                        idx_bufs[t][pl.ds(gl * G, L)],
                        pos_bufs[t][pl.ds(gl * G, L)])
                    idx_gs[b][...] = kv[0]
                    pos_gs[b][...] = kv[1]
                    pltpu.make_async_copy(
                        tables[t].at[idx_gs[b]],
                        rows[b], gsems[b]).start()
                return br

            lax.switch(t_sel, [mk_branch(t) for t in range(NUM_TABLES)])

        def gwait(b):
            pltpu.make_async_copy(
                tables[0].at[idx_gs[b]],
                rows[b], gsems[b]).wait()

        def scopy(b):
            return pltpu.make_async_copy(
                rows[b], out_hbm.at[pos_gs[b]], ssems[b])

        ns = (ngt + (NBUF - 1)) // NBUF

        def sbody(s, carry):
            for b in range(NBUF):
                g = s * NBUF + b

                @pl.when(s > 0)
                def _(b=b):
                    scopy(b).wait()

                @pl.when(g < ngt)
                def _(b=b, g=g):
                    start_group(b, g)

            for b in range(NBUF):
                g = s * NBUF + b

                @pl.when(g < ngt)
                def _(b=b):
                    gwait(b)
                    scopy(b).start()
            return carry

        lax.fori_loop(0, ns, sbody, jnp.int32(0))
        for b in range(NBUF):
            @pl.when((ns - 1) * NBUF + b < ngt)
            def _(b=b):
                scopy(b).wait()

    return body(ids_flat, text_w, audio_w, special_w, phoneme_w, reserved_w)


def kernel(input_ids, text_w, audio_w, special_w, phoneme_w, reserved_w):
    shape = input_ids.shape
    ids_flat = input_ids.reshape(-1).astype(jnp.int32)
    out = _embed(ids_flat, text_w, audio_w, special_w, phoneme_w, reserved_w)
    return out.reshape(*shape, D_MODEL)
